# Initial kernel scaffold; baseline (speedup 1.0000x reference)
import functools

import jax
import jax.numpy as jnp
from jax import lax
from jax.experimental import pallas as pl
from jax.experimental.pallas import tpu as pltpu
from jax.experimental.pallas import tpu_sc as plsc

N = 10000
E = 320000
NP = 10240
NC = 2
NS = 16
NW = NC * NS
EPW = E // NW
CH = 80
RPT = NP // NS
F32 = jnp.float32


def _tc_stage(outA, outB, denA, denB, ex, resin, wres, bias8, W=None,
              aal=None, aar=None):
    with_proj = W is not None
    dh = outA.shape[1]
    dr = resin.shape[1]
    B = 400
    grid = (N // B,)

    def body(*refs):
        if with_proj:
            (oA, oB, dA, dB, exr, rin, wr, br, Wr, alr, arr,
             h_o, f_o, l_o, r_o) = refs
        else:
            oA, oB, dA, dB, exr, rin, wr, br, h_o = refs
        den = dA[...] + dB[...]
        rec = 1.0 / jnp.maximum(den, 1e-9)
        recx = jnp.dot(rec, exr[...], preferred_element_type=F32)
        h = (oA[...] + oB[...]) * recx
        h = h + jnp.dot(rin[...], wr[...], preferred_element_type=F32)
        h = h + br[0:1, :]
        h_o[...] = h
        if with_proj:
            f = jnp.dot(h, Wr[...], preferred_element_type=F32)
            f_o[...] = f
            l_o[...] = jnp.dot(f, alr[...], preferred_element_type=F32)
            r_o[...] = jnp.dot(f, arr[...], preferred_element_type=F32)

    node = lambda d: pl.BlockSpec((B, d), lambda i: (i, 0))
    full = lambda a: pl.BlockSpec(a.shape, lambda i: (0, 0))
    in_specs = [node(dh), node(dh), node(16), node(16), full(ex),
                node(dr), full(wres), full(bias8)]
    args = [outA, outB, denA, denB, ex, resin, wres, bias8]
    out_shapes = [jax.ShapeDtypeStruct((N, dh), F32)]
    out_specs = [node(dh)]
    if with_proj:
        dn = W.shape[1]
        in_specs += [full(W), full(aal), full(aar)]
        args += [W, aal, aar]
        out_shapes += [jax.ShapeDtypeStruct((N, dn), F32),
                       jax.ShapeDtypeStruct((N, 16), F32),
                       jax.ShapeDtypeStruct((N, 16), F32)]
        out_specs += [node(dn), node(16), node(16)]
    return pl.pallas_call(
        body, grid=grid, in_specs=in_specs, out_specs=out_specs,
        out_shape=out_shapes)(*args)


def _sc_pass(src, dst, L, R, F, zf, zd, headmap):
    DW = F.shape[1]
    mesh = plsc.VectorSubcoreMesh(core_axis_name="c", subcore_axis_name="s")

    @functools.partial(
        pl.kernel,
        out_type=[jax.ShapeDtypeStruct((NC, NP, DW), F32),
                  jax.ShapeDtypeStruct((NC, NP, 16), F32)],
        mesh=mesh,
        scratch_types=[
            pltpu.VMEM_SHARED((NP, DW), F32),
            pltpu.VMEM_SHARED((NP, 16), F32),
            pltpu.VMEM((CH,), jnp.int32),
            pltpu.VMEM((CH,), jnp.int32),
            pltpu.VMEM((CH, 16), F32),
            pltpu.VMEM((CH, 16), F32),
            pltpu.VMEM((CH, DW), F32),
            pltpu.VMEM((CH, 16), F32),
            pltpu.SemaphoreType.DMA,
            pltpu.SemaphoreType.DMA,
            pltpu.SemaphoreType.DMA,
        ],
    )
    def k(src_r, dst_r, L_r, R_r, F_r, zf_r, zd_r, outp_r, denp_r,
          out_sp, den_sp, sidx, didx, lrow, rrow, frow, wrow,
          sem0, sem1, sem2):
        c = lax.axis_index("c")
        s = lax.axis_index("s")
        g = s * NC + c
        rows0 = s * RPT
        pltpu.sync_copy(zf_r.at[pl.ds(rows0, RPT)],
                        out_sp.at[pl.ds(rows0, RPT)])
        pltpu.sync_copy(zd_r.at[pl.ds(rows0, RPT)],
                        den_sp.at[pl.ds(rows0, RPT)])
        plsc.subcore_barrier()

        base = g * EPW

        @pl.loop(0, EPW, step=CH)
        def _chunk(k0):
            b = base + k0
            pltpu.sync_copy(src_r.at[pl.ds(b, CH)], sidx)
            pltpu.sync_copy(dst_r.at[pl.ds(b, CH)], didx)
            cl = pltpu.async_copy(L_r.at[sidx], lrow, sem0)
            cr = pltpu.async_copy(R_r.at[didx], rrow, sem1)
            cf = pltpu.async_copy(F_r.at[sidx], frow, sem2)
            cl.wait()
            cr.wait()
            cf.wait()

            @pl.loop(0, CH)
            def _edge(e):
                x = lrow[e, :] + rrow[e, :]
                x = jnp.maximum(x, x * 0.2)
                w = jnp.exp(x)
                wrow[e, :] = w
                for j, hj in enumerate(headmap):
                    frow[e, pl.ds(16 * j, 16)] = (
                        frow[e, pl.ds(16 * j, 16)] * w[hj])

            pltpu.sync_copy(wrow, den_sp.at[didx], add=True)
            pltpu.sync_copy(frow, out_sp.at[didx], add=True)

        plsc.subcore_barrier()
        pltpu.sync_copy(out_sp.at[pl.ds(rows0, RPT)],
                        outp_r.at[c, pl.ds(rows0, RPT)])
        pltpu.sync_copy(den_sp.at[pl.ds(rows0, RPT)],
                        denp_r.at[c, pl.ds(rows0, RPT)])

    return k(src, dst, L, R, F, zf, zd)


def _att_mat(a, din):
    H, D = a.shape
    m = jnp.zeros((din, 16), F32)
    for h in range(H):
        m = m.at[h * D:(h + 1) * D, h].set(a[h])
    return m


def _ex_mat(dh, H, D):
    m = jnp.zeros((16, dh), F32)
    for h in range(H):
        m = m.at[h, h * D:(h + 1) * D].set(1.0)
    return m


def _pad_rows(x):
    return jnp.pad(x, ((0, NP - N), (0, 0)))


def kernel(edge_index, features, W0, al0, ar0, b0, W1, al1, ar1, b1,
           W2, al2, ar2, b2, resW2):
    ei = edge_index.astype(jnp.int32)
    src, dst = ei[0], ei[1]

    I128 = jnp.eye(128, dtype=F32)
    zn128 = jnp.zeros((N, 128), F32)
    zn16 = jnp.zeros((N, 16), F32)
    zf128 = jnp.zeros((NP, 128), F32)
    zf48 = jnp.zeros((NP, 48), F32)
    zd16 = jnp.zeros((NP, 16), F32)
    z8_128 = jnp.zeros((8, 128), F32)
    ex128 = _ex_mat(128, 8, 16)
    ex48 = _ex_mat(48, 1, 48)
    heads8 = tuple(range(8))
    heads1 = (0, 0, 0)

    _, f0, L0, R0 = _tc_stage(zn128, zn128, zn16, zn16, ex128, features,
                              I128, z8_128, W0,
                              _att_mat(al0, 128), _att_mat(ar0, 128))
    o0, d0 = _sc_pass(src, dst, _pad_rows(L0), _pad_rows(R0),
                      _pad_rows(f0), zf128, zd16, heads8)

    b0t = jnp.tile(b0.reshape(1, 128), (8, 1))
    h1, f1, L1, R1 = _tc_stage(o0[0, :N], o0[1, :N], d0[0, :N, :],
                               d0[1, :N, :], ex128, zn128, I128, b0t, W1,
                               _att_mat(al1, 128), _att_mat(ar1, 128))
    o1, d1 = _sc_pass(src, dst, _pad_rows(L1), _pad_rows(R1),
                      _pad_rows(f1), zf128, zd16, heads8)

    b1t = jnp.tile(b1.reshape(1, 128), (8, 1))
    W2p = jnp.pad(W2, ((0, 0), (0, 8)))
    al2m = jnp.zeros((48, 16), F32).at[0:40, 0].set(al2[0])
    ar2m = jnp.zeros((48, 16), F32).at[0:40, 0].set(ar2[0])
    h2, f2, L2, R2 = _tc_stage(o1[0, :N], o1[1, :N], d1[0, :N, :],
                               d1[1, :N, :], ex128, h1, I128, b1t, W2p,
                               al2m, ar2m)
    o2, d2 = _sc_pass(src, dst, _pad_rows(L2), _pad_rows(R2),
                      _pad_rows(f2), zf48, zd16, heads1)

    resW2p = jnp.pad(resW2, ((0, 0), (0, 8)))
    b2t = jnp.tile(jnp.pad(b2, (0, 8)).reshape(1, 48), (8, 1))
    (h3,) = _tc_stage(o2[0, :N], o2[1, :N], d2[0, :N, :], d2[1, :N, :],
                      ex48, h2, resW2p, b2t)
    return h3[:, :40]


# speedup vs baseline: 30.2496x; 30.2496x over previous
import functools

import jax
import jax.numpy as jnp
from jax import lax
from jax.experimental import pallas as pl
from jax.experimental.pallas import tpu as pltpu
from jax.experimental.pallas import tpu_sc as plsc

N = 10000
E = 320000
NP = 10240
NC = 2
NS = 16
NW = NC * NS
EPW = E // NW
CH = 80
F32 = jnp.float32


def _tc_stage(outA, outB, denA, denB, ex, resin, wres, bias8, W=None,
              aal=None, aar=None):
    with_proj = W is not None
    dh = outA.shape[1]
    dr = resin.shape[1]
    B = 400
    grid = (N // B,)

    def body(*refs):
        if with_proj:
            (oA, oB, dA, dB, exr, rin, wr, br, Wr, alr, arr,
             h_o, f_o, l_o, r_o) = refs
        else:
            oA, oB, dA, dB, exr, rin, wr, br, h_o = refs
        den = dA[...] + dB[...]
        rec = 1.0 / jnp.maximum(den, 1e-9)
        recx = jnp.dot(rec, exr[...], preferred_element_type=F32)
        h = (oA[...] + oB[...]) * recx
        h = h + jnp.dot(rin[...], wr[...], preferred_element_type=F32)
        h = h + br[0:1, :]
        h_o[...] = h
        if with_proj:
            f = jnp.dot(h, Wr[...], preferred_element_type=F32)
            f_o[...] = f
            l_o[...] = jnp.dot(f, alr[...], preferred_element_type=F32)
            r_o[...] = jnp.dot(f, arr[...], preferred_element_type=F32)

    node = lambda d: pl.BlockSpec((B, d), lambda i: (i, 0))
    full = lambda a: pl.BlockSpec(a.shape, lambda i: (0, 0))
    in_specs = [node(dh), node(dh), node(16), node(16), full(ex),
                node(dr), full(wres), full(bias8)]
    args = [outA, outB, denA, denB, ex, resin, wres, bias8]
    out_shapes = [jax.ShapeDtypeStruct((N, dh), F32)]
    out_specs = [node(dh)]
    if with_proj:
        dn = W.shape[1]
        in_specs += [full(W), full(aal), full(aar)]
        args += [W, aal, aar]
        out_shapes += [jax.ShapeDtypeStruct((N, dn), F32),
                       jax.ShapeDtypeStruct((N, 16), F32),
                       jax.ShapeDtypeStruct((N, 16), F32)]
        out_specs += [node(dn), node(16), node(16)]
    return pl.pallas_call(
        body, grid=grid, in_specs=in_specs, out_specs=out_specs,
        out_shape=out_shapes)(*args)


def _sc_pass(src, dst, L, R, F, zacc, zden, headmap):
    DW = F.shape[1]
    npiece = DW // 16
    lpe = npiece
    tr = CH * npiece
    blen = {8: 128, 4: 80}[npiece]
    nbatch = tr // blen
    vpb = blen // 16
    epv = 16 // npiece
    rpt_a = NP * npiece // NS
    rpt_d = NP // NS
    mesh = plsc.VectorSubcoreMesh(core_axis_name="c", subcore_axis_name="s")

    @functools.partial(
        pl.kernel,
        out_type=[jax.ShapeDtypeStruct((NC, NP * npiece, 16), F32),
                  jax.ShapeDtypeStruct((NC, NP, 16), F32)],
        mesh=mesh,
        compiler_params=pltpu.CompilerParams(use_tc_tiling_on_sc=False),
        scratch_types=[
            pltpu.VMEM_SHARED((NP * npiece, 16), F32),
            pltpu.VMEM_SHARED((NP, 16), F32),
            pltpu.VMEM((CH,), jnp.int32),
            pltpu.VMEM((CH,), jnp.int32),
            pltpu.VMEM((CH, 16), F32),
            pltpu.VMEM((CH, 16), F32),
            pltpu.VMEM((CH, DW), F32),
            pltpu.VMEM((CH, 16), F32),
            pltpu.VMEM((tr, 16), F32),
            [pltpu.VMEM((blen,), jnp.int32) for _ in range(nbatch)],
            pltpu.SemaphoreType.DMA,
            pltpu.SemaphoreType.DMA,
            pltpu.SemaphoreType.DMA,
        ],
    )
    def k(src_r, dst_r, L_r, R_r, F_r, zacc_r, zden_r, accp_r, denp_r,
          acc_sp, den_sp, sidx, didx, lrow, rrow, frow, wrow, fs, idxbs,
          sem0, sem1, sem2):
        c = lax.axis_index("c")
        s = lax.axis_index("s")
        g = s * NC + c
        ra = s * rpt_a
        rd = s * rpt_d
        pltpu.sync_copy(zacc_r.at[pl.ds(ra, rpt_a)],
                        acc_sp.at[pl.ds(ra, rpt_a)])
        pltpu.sync_copy(zden_r.at[pl.ds(rd, rpt_d)],
                        den_sp.at[pl.ds(rd, rpt_d)])
        plsc.subcore_barrier()

        base = g * EPW
        io = lax.iota(jnp.int32, 16)
        jadd = io & (npiece - 1)

        @pl.loop(0, EPW, step=CH)
        def _chunk(k0):
            b = base + k0
            pltpu.sync_copy(src_r.at[pl.ds(b, CH)], sidx)
            pltpu.sync_copy(dst_r.at[pl.ds(b, CH)], didx)
            cl = pltpu.async_copy(L_r.at[sidx], lrow, sem0)
            cr = pltpu.async_copy(R_r.at[didx], rrow, sem1)
            cf = pltpu.async_copy(F_r.at[sidx], frow, sem2)
            cl.wait()
            cr.wait()
            cf.wait()

            dvs = [didx[pl.ds(16 * q, 16)] for q in range(CH // 16)]
            for v in range(tr // 16):
                e0 = v * epv
                dv = dvs[e0 // 16]
                lane0 = e0 % 16
                d = dv[lane0]
                for t in range(1, epv):
                    d = jnp.where(io < t * lpe, d, dv[lane0 + t])
                vec = d * npiece + jadd
                idxbs[v // vpb][pl.ds(16 * (v % vpb), 16)] = vec

            @pl.loop(0, CH)
            def _edge(e):
                x = lrow[e, :] + rrow[e, :]
                x = jnp.maximum(x, x * 0.2)
                w = jnp.exp(x)
                wrow[e, :] = w
                for j, hj in enumerate(headmap):
                    fs[e * npiece + j, :] = frow[e, pl.ds(16 * j, 16)] * w[hj]

            pltpu.sync_copy(wrow, den_sp.at[didx], add=True)
            for bi in range(nbatch):
                pltpu.sync_copy(fs.at[pl.ds(blen * bi, blen)],
                                acc_sp.at[idxbs[bi]], add=True)

        plsc.subcore_barrier()
        pltpu.sync_copy(acc_sp.at[pl.ds(ra, rpt_a)],
                        accp_r.at[c, pl.ds(ra, rpt_a)])
        pltpu.sync_copy(den_sp.at[pl.ds(rd, rpt_d)],
                        denp_r.at[c, pl.ds(rd, rpt_d)])

    return k(src, dst, L, R, F, zacc, zden)


def _att_mat(a, din):
    H, D = a.shape
    m = jnp.zeros((din, 16), F32)
    for h in range(H):
        m = m.at[h * D:(h + 1) * D, h].set(a[h])
    return m


def _ex_mat(dh, H, D):
    m = jnp.zeros((16, dh), F32)
    for h in range(H):
        m = m.at[h, h * D:(h + 1) * D].set(1.0)
    return m


def _pad_rows(x):
    return jnp.pad(x, ((0, NP - N), (0, 0)))


def kernel(edge_index, features, W0, al0, ar0, b0, W1, al1, ar1, b1,
           W2, al2, ar2, b2, resW2):
    ei = edge_index.astype(jnp.int32)
    src, dst = ei[0], ei[1]

    I128 = jnp.eye(128, dtype=F32)
    zn128 = jnp.zeros((N, 128), F32)
    zn16 = jnp.zeros((N, 16), F32)
    zacc8 = jnp.zeros((NP * 8, 16), F32)
    zacc4 = jnp.zeros((NP * 4, 16), F32)
    zden = jnp.zeros((NP, 16), F32)
    z8_128 = jnp.zeros((8, 128), F32)
    ex128 = _ex_mat(128, 8, 16)
    ex64 = _ex_mat(64, 1, 64)
    heads8 = tuple(range(8))
    heads1 = (0, 0, 0, 0)

    _, f0, L0, R0 = _tc_stage(zn128, zn128, zn16, zn16, ex128, features,
                              I128, z8_128, W0,
                              _att_mat(al0, 128), _att_mat(ar0, 128))
    a0, d0 = _sc_pass(src, dst, _pad_rows(L0), _pad_rows(R0),
                      _pad_rows(f0), zacc8, zden, heads8)

    b0t = jnp.tile(b0.reshape(1, 128), (8, 1))
    o0A = a0[0].reshape(NP, 128)[:N]
    o0B = a0[1].reshape(NP, 128)[:N]
    h1, f1, L1, R1 = _tc_stage(o0A, o0B, d0[0, :N, :], d0[1, :N, :],
                               ex128, zn128, I128, b0t, W1,
                               _att_mat(al1, 128), _att_mat(ar1, 128))
    a1, d1 = _sc_pass(src, dst, _pad_rows(L1), _pad_rows(R1),
                      _pad_rows(f1), zacc8, zden, heads8)

    b1t = jnp.tile(b1.reshape(1, 128), (8, 1))
    W2p = jnp.pad(W2, ((0, 0), (0, 24)))
    al2m = jnp.zeros((64, 16), F32).at[0:40, 0].set(al2[0])
    ar2m = jnp.zeros((64, 16), F32).at[0:40, 0].set(ar2[0])
    o1A = a1[0].reshape(NP, 128)[:N]
    o1B = a1[1].reshape(NP, 128)[:N]
    h2, f2, L2, R2 = _tc_stage(o1A, o1B, d1[0, :N, :], d1[1, :N, :],
                               ex128, h1, I128, b1t, W2p, al2m, ar2m)
    a2, d2 = _sc_pass(src, dst, _pad_rows(L2), _pad_rows(R2),
                      _pad_rows(f2), zacc4, zden, heads1)

    resW2p = jnp.pad(resW2, ((0, 0), (0, 24)))
    b2t = jnp.tile(jnp.pad(b2, (0, 24)).reshape(1, 64), (8, 1))
    o2A = a2[0].reshape(NP, 64)[:N]
    o2B = a2[1].reshape(NP, 64)[:N]
    (h3,) = _tc_stage(o2A, o2B, d2[0, :N, :], d2[1, :N, :],
                      ex64, h2, resW2p, b2t)
    return h3[:, :40]


# speedup vs baseline: 31.1671x; 1.0303x over previous
import functools

import jax
import jax.numpy as jnp
from jax import lax
from jax.experimental import pallas as pl
from jax.experimental.pallas import tpu as pltpu
from jax.experimental.pallas import tpu_sc as plsc

N = 10000
E = 320000
NP = 10240
NC = 2
NS = 16
NW = NC * NS
EPW = E // NW
CH = 80
F32 = jnp.float32


def _tc_stage(outA, outB, denA, denB, ex, resin, wres, bias8, W=None,
              aal=None, aar=None):
    with_proj = W is not None
    dh = outA.shape[1]
    dr = resin.shape[1]
    B = 400
    grid = (N // B,)

    def body(*refs):
        if with_proj:
            (oA, oB, dA, dB, exr, rin, wr, br, Wr, alr, arr,
             h_o, f_o, l_o, r_o) = refs
        else:
            oA, oB, dA, dB, exr, rin, wr, br, h_o = refs
        den = dA[...] + dB[...]
        rec = 1.0 / jnp.maximum(den, 1e-9)
        recx = jnp.dot(rec, exr[...], preferred_element_type=F32)
        h = (oA[...] + oB[...]) * recx
        h = h + jnp.dot(rin[...], wr[...], preferred_element_type=F32)
        h = h + br[0:1, :]
        h_o[...] = h
        if with_proj:
            f = jnp.dot(h, Wr[...], preferred_element_type=F32)
            f_o[...] = f
            l_o[...] = jnp.dot(f, alr[...], preferred_element_type=F32)
            r_o[...] = jnp.dot(f, arr[...], preferred_element_type=F32)

    node = lambda d: pl.BlockSpec((B, d), lambda i: (i, 0))
    full = lambda a: pl.BlockSpec(a.shape, lambda i: (0, 0))
    in_specs = [node(dh), node(dh), node(16), node(16), full(ex),
                node(dr), full(wres), full(bias8)]
    args = [outA, outB, denA, denB, ex, resin, wres, bias8]
    out_shapes = [jax.ShapeDtypeStruct((N, dh), F32)]
    out_specs = [node(dh)]
    if with_proj:
        dn = W.shape[1]
        in_specs += [full(W), full(aal), full(aar)]
        args += [W, aal, aar]
        out_shapes += [jax.ShapeDtypeStruct((N, dn), F32),
                       jax.ShapeDtypeStruct((N, 16), F32),
                       jax.ShapeDtypeStruct((N, 16), F32)]
        out_specs += [node(dn), node(16), node(16)]
    return pl.pallas_call(
        body, grid=grid, in_specs=in_specs, out_specs=out_specs,
        out_shape=out_shapes)(*args)


def _sc_pass(src, dst, L, R, F, zacc, zden, headmap):
    DW = F.shape[1]
    npiece = DW // 16
    lpe = npiece
    tr = CH * npiece
    blen = {8: 128, 4: 80}[npiece]
    nbatch = tr // blen
    vpb = blen // 16
    epv = 16 // npiece
    rpt_a = NP * npiece // NS
    rpt_d = NP // NS
    mesh = plsc.VectorSubcoreMesh(core_axis_name="c", subcore_axis_name="s")

    @functools.partial(
        pl.kernel,
        out_type=[jax.ShapeDtypeStruct((NC, NP * npiece, 16), F32),
                  jax.ShapeDtypeStruct((NC, NP, 16), F32)],
        mesh=mesh,
        compiler_params=pltpu.CompilerParams(use_tc_tiling_on_sc=False),
        scratch_types=[
            pltpu.VMEM_SHARED((NP * npiece, 16), F32),
            pltpu.VMEM_SHARED((NP, 16), F32),
            pltpu.VMEM((CH,), jnp.int32),
            pltpu.VMEM((CH,), jnp.int32),
            pltpu.VMEM((CH, 16), F32),
            pltpu.VMEM((CH, 16), F32),
            pltpu.VMEM((CH, DW), F32),
            pltpu.VMEM((CH, 16), F32),
            pltpu.VMEM((tr, 16), F32),
            pltpu.VMEM((tr,), jnp.int32),
            pltpu.SemaphoreType.DMA,
            pltpu.SemaphoreType.DMA,
            pltpu.SemaphoreType.DMA,
        ],
    )
    def k(src_r, dst_r, L_r, R_r, F_r, zacc_r, zden_r, accp_r, denp_r,
          acc_sp, den_sp, sidx, didx, lrow, rrow, frow, wrow, fs, idxb,
          sem0, sem1, sem2):
        c = lax.axis_index("c")
        s = lax.axis_index("s")
        g = s * NC + c
        ra = s * rpt_a
        rd = s * rpt_d
        pltpu.sync_copy(zacc_r.at[pl.ds(ra, rpt_a)],
                        acc_sp.at[pl.ds(ra, rpt_a)])
        pltpu.sync_copy(zden_r.at[pl.ds(rd, rpt_d)],
                        den_sp.at[pl.ds(rd, rpt_d)])
        plsc.subcore_barrier()

        base = g * EPW
        io = lax.iota(jnp.int32, 16)
        jadd = io & (npiece - 1)

        @pl.loop(0, EPW, step=CH)
        def _chunk(k0):
            b = base + k0
            pltpu.sync_copy(src_r.at[pl.ds(b, CH)], sidx)
            pltpu.sync_copy(dst_r.at[pl.ds(b, CH)], didx)
            cl = pltpu.async_copy(L_r.at[sidx], lrow, sem0)
            cr = pltpu.async_copy(R_r.at[didx], rrow, sem1)
            cf = pltpu.async_copy(F_r.at[sidx], frow, sem2)
            cl.wait()
            cr.wait()
            cf.wait()

            dvs = [didx[pl.ds(16 * q, 16)] for q in range(CH // 16)]
            for v in range(tr // 16):
                e0 = v * epv
                dv = dvs[e0 // 16]
                lane0 = e0 % 16
                d = dv[lane0]
                for t in range(1, epv):
                    d = jnp.where(io < t * lpe, d, dv[lane0 + t])
                vec = d * npiece + jadd
                idxb[pl.ds(16 * v, 16)] = vec

            @pl.loop(0, CH)
            def _edge(e):
                x = lrow[e, :] + rrow[e, :]
                x = jnp.maximum(x, x * 0.2)
                w = jnp.exp(x)
                wrow[e, :] = w
                for j, hj in enumerate(headmap):
                    fs[e * npiece + j, :] = frow[e, pl.ds(16 * j, 16)] * w[hj]

            pltpu.sync_copy(wrow, den_sp.at[didx], add=True)
            pltpu.sync_copy(fs, acc_sp.at[idxb], add=True)

        plsc.subcore_barrier()
        pltpu.sync_copy(acc_sp.at[pl.ds(ra, rpt_a)],
                        accp_r.at[c, pl.ds(ra, rpt_a)])
        pltpu.sync_copy(den_sp.at[pl.ds(rd, rpt_d)],
                        denp_r.at[c, pl.ds(rd, rpt_d)])

    return k(src, dst, L, R, F, zacc, zden)


def _att_mat(a, din):
    H, D = a.shape
    m = jnp.zeros((din, 16), F32)
    for h in range(H):
        m = m.at[h * D:(h + 1) * D, h].set(a[h])
    return m


def _ex_mat(dh, H, D):
    m = jnp.zeros((16, dh), F32)
    for h in range(H):
        m = m.at[h, h * D:(h + 1) * D].set(1.0)
    return m


def _pad_rows(x):
    return jnp.pad(x, ((0, NP - N), (0, 0)))


def kernel(edge_index, features, W0, al0, ar0, b0, W1, al1, ar1, b1,
           W2, al2, ar2, b2, resW2):
    ei = edge_index.astype(jnp.int32)
    src, dst = ei[0], ei[1]

    I128 = jnp.eye(128, dtype=F32)
    zn128 = jnp.zeros((N, 128), F32)
    zn16 = jnp.zeros((N, 16), F32)
    zacc8 = jnp.zeros((NP * 8, 16), F32)
    zacc4 = jnp.zeros((NP * 4, 16), F32)
    zden = jnp.zeros((NP, 16), F32)
    z8_128 = jnp.zeros((8, 128), F32)
    ex128 = _ex_mat(128, 8, 16)
    ex64 = _ex_mat(64, 1, 64)
    heads8 = tuple(range(8))
    heads1 = (0, 0, 0, 0)

    _, f0, L0, R0 = _tc_stage(zn128, zn128, zn16, zn16, ex128, features,
                              I128, z8_128, W0,
                              _att_mat(al0, 128), _att_mat(ar0, 128))
    a0, d0 = _sc_pass(src, dst, _pad_rows(L0), _pad_rows(R0),
                      _pad_rows(f0), zacc8, zden, heads8)

    b0t = jnp.tile(b0.reshape(1, 128), (8, 1))
    o0A = a0[0].reshape(NP, 128)[:N]
    o0B = a0[1].reshape(NP, 128)[:N]
    h1, f1, L1, R1 = _tc_stage(o0A, o0B, d0[0, :N, :], d0[1, :N, :],
                               ex128, zn128, I128, b0t, W1,
                               _att_mat(al1, 128), _att_mat(ar1, 128))
    a1, d1 = _sc_pass(src, dst, _pad_rows(L1), _pad_rows(R1),
                      _pad_rows(f1), zacc8, zden, heads8)

    b1t = jnp.tile(b1.reshape(1, 128), (8, 1))
    W2p = jnp.pad(W2, ((0, 0), (0, 24)))
    al2m = jnp.zeros((64, 16), F32).at[0:40, 0].set(al2[0])
    ar2m = jnp.zeros((64, 16), F32).at[0:40, 0].set(ar2[0])
    o1A = a1[0].reshape(NP, 128)[:N]
    o1B = a1[1].reshape(NP, 128)[:N]
    h2, f2, L2, R2 = _tc_stage(o1A, o1B, d1[0, :N, :], d1[1, :N, :],
                               ex128, h1, I128, b1t, W2p, al2m, ar2m)
    a2, d2 = _sc_pass(src, dst, _pad_rows(L2), _pad_rows(R2),
                      _pad_rows(f2), zacc4, zden, heads1)

    resW2p = jnp.pad(resW2, ((0, 0), (0, 24)))
    b2t = jnp.tile(jnp.pad(b2, (0, 24)).reshape(1, 64), (8, 1))
    o2A = a2[0].reshape(NP, 64)[:N]
    o2B = a2[1].reshape(NP, 64)[:N]
    (h3,) = _tc_stage(o2A, o2B, d2[0, :N, :], d2[1, :N, :],
                      ex64, h2, resW2p, b2t)
    return h3[:, :40]


# speedup vs baseline: 36.1321x; 1.1593x over previous
import functools

import jax
import jax.numpy as jnp
from jax import lax
from jax.experimental import pallas as pl
from jax.experimental.pallas import tpu as pltpu
from jax.experimental.pallas import tpu_sc as plsc

N = 10000
E = 320000
NP = 10240
NC = 2
NS = 16
NW = NC * NS
EPW = E // NW
CH = 80
F32 = jnp.float32


def _tc_stage(outA, outB, denA, denB, ex, resin, wres, bias8, W=None,
              aal=None, aar=None):
    with_proj = W is not None
    dh = outA.shape[1]
    dr = resin.shape[1]
    B = 400
    grid = (N // B,)

    def body(*refs):
        if with_proj:
            (oA, oB, dA, dB, exr, rin, wr, br, Wr, alr, arr,
             h_o, f_o, l_o, r_o) = refs
        else:
            oA, oB, dA, dB, exr, rin, wr, br, h_o = refs
        den = dA[...] + dB[...]
        rec = 1.0 / jnp.maximum(den, 1e-9)
        recx = jnp.dot(rec, exr[...], preferred_element_type=F32)
        h = (oA[...] + oB[...]) * recx
        h = h + jnp.dot(rin[...], wr[...], preferred_element_type=F32)
        h = h + br[0:1, :]
        h_o[...] = h
        if with_proj:
            f = jnp.dot(h, Wr[...], preferred_element_type=F32)
            f_o[...] = f
            l_o[...] = jnp.dot(f, alr[...], preferred_element_type=F32)
            r_o[...] = jnp.dot(f, arr[...], preferred_element_type=F32)

    node = lambda d: pl.BlockSpec((B, d), lambda i: (i, 0))
    full = lambda a: pl.BlockSpec(a.shape, lambda i: (0, 0))
    in_specs = [node(dh), node(dh), node(16), node(16), full(ex),
                node(dr), full(wres), full(bias8)]
    args = [outA, outB, denA, denB, ex, resin, wres, bias8]
    out_shapes = [jax.ShapeDtypeStruct((N, dh), F32)]
    out_specs = [node(dh)]
    if with_proj:
        dn = W.shape[1]
        in_specs += [full(W), full(aal), full(aar)]
        args += [W, aal, aar]
        out_shapes += [jax.ShapeDtypeStruct((N, dn), F32),
                       jax.ShapeDtypeStruct((N, 16), F32),
                       jax.ShapeDtypeStruct((N, 16), F32)]
        out_specs += [node(dn), node(16), node(16)]
    return pl.pallas_call(
        body, grid=grid, in_specs=in_specs, out_specs=out_specs,
        out_shape=out_shapes)(*args)


def _sc_pass(src, dst, L, R, F, zacc, zden, headmap):
    DW = F.shape[1]
    npiece = DW // 16
    lpe = npiece
    tr = CH * npiece
    blen = {8: 128, 4: 80}[npiece]
    nbatch = tr // blen
    vpb = blen // 16
    epv = 16 // npiece
    rpt_a = NP * npiece // NS
    rpt_d = NP // NS
    mesh = plsc.VectorSubcoreMesh(core_axis_name="c", subcore_axis_name="s")

    @functools.partial(
        pl.kernel,
        out_type=[jax.ShapeDtypeStruct((NC, NP * npiece, 16), F32),
                  jax.ShapeDtypeStruct((NC, NP, 16), F32)],
        mesh=mesh,
        compiler_params=pltpu.CompilerParams(use_tc_tiling_on_sc=False),
        scratch_types=[
            pltpu.VMEM_SHARED((NP * npiece, 16), F32),
            pltpu.VMEM_SHARED((NP, 16), F32),
            [pltpu.VMEM((CH,), jnp.int32) for _ in range(2)],
            [pltpu.VMEM((CH,), jnp.int32) for _ in range(2)],
            [pltpu.VMEM((CH, 16), F32) for _ in range(2)],
            [pltpu.VMEM((CH, 16), F32) for _ in range(2)],
            [pltpu.VMEM((CH, DW), F32) for _ in range(2)],
            pltpu.VMEM((CH, 16), F32),
            pltpu.VMEM((tr, 16), F32),
            pltpu.VMEM((tr,), jnp.int32),
            [pltpu.SemaphoreType.DMA for _ in range(2)],
        ],
    )
    def k(src_r, dst_r, L_r, R_r, F_r, zacc_r, zden_r, accp_r, denp_r,
          acc_sp, den_sp, sidxs, didxs, lrows, rrows, frows, wrow, fs,
          idxb, semg):
        c = lax.axis_index("c")
        s = lax.axis_index("s")
        g = s * NC + c
        ra = s * rpt_a
        rd = s * rpt_d
        pltpu.sync_copy(zacc_r.at[pl.ds(ra, rpt_a)],
                        acc_sp.at[pl.ds(ra, rpt_a)])
        pltpu.sync_copy(zden_r.at[pl.ds(rd, rpt_d)],
                        den_sp.at[pl.ds(rd, rpt_d)])
        plsc.subcore_barrier()

        base = g * EPW
        nchunk = EPW // CH
        io = lax.iota(jnp.int32, 16)
        jadd = io & (npiece - 1)

        def load_idx(i, b):
            off = base + i * CH
            pltpu.sync_copy(src_r.at[pl.ds(off, CH)], sidxs[b])
            pltpu.sync_copy(dst_r.at[pl.ds(off, CH)], didxs[b])

        def start_g(b):
            pltpu.async_copy(L_r.at[sidxs[b]], lrows[b], semg[b])
            pltpu.async_copy(R_r.at[didxs[b]], rrows[b], semg[b])
            pltpu.async_copy(F_r.at[sidxs[b]], frows[b], semg[b])

        def wait_g(b):
            pltpu.make_async_copy(L_r.at[sidxs[b]], lrows[b], semg[b]).wait()
            pltpu.make_async_copy(R_r.at[didxs[b]], rrows[b], semg[b]).wait()
            pltpu.make_async_copy(F_r.at[sidxs[b]], frows[b], semg[b]).wait()

        def process(b):
            didx, lrow, rrow, frow = didxs[b], lrows[b], rrows[b], frows[b]
            dvs = [didx[pl.ds(16 * q, 16)] for q in range(CH // 16)]
            for v in range(tr // 16):
                e0 = v * epv
                dv = dvs[e0 // 16]
                lane0 = e0 % 16
                d = dv[lane0]
                for t in range(1, epv):
                    d = jnp.where(io < t * lpe, d, dv[lane0 + t])
                vec = d * npiece + jadd
                idxb[pl.ds(16 * v, 16)] = vec

            @pl.loop(0, CH, unroll=4)
            def _edge(e):
                x = lrow[e, :] + rrow[e, :]
                x = jnp.maximum(x, x * 0.2)
                w = jnp.exp(x)
                wrow[e, :] = w
                for j, hj in enumerate(headmap):
                    fs[e * npiece + j, :] = frow[e, pl.ds(16 * j, 16)] * w[hj]

            pltpu.sync_copy(wrow, den_sp.at[didx], add=True)
            pltpu.sync_copy(fs, acc_sp.at[idxb], add=True)

        load_idx(0, 0)
        start_g(0)

        @pl.loop(0, nchunk - 1, step=2)
        def _pair(i):
            load_idx(i + 1, 1)
            start_g(1)
            wait_g(0)
            process(0)
            load_idx(i + 2, 0)
            start_g(0)
            wait_g(1)
            process(1)

        wait_g(0)
        process(0)

        plsc.subcore_barrier()
        pltpu.sync_copy(acc_sp.at[pl.ds(ra, rpt_a)],
                        accp_r.at[c, pl.ds(ra, rpt_a)])
        pltpu.sync_copy(den_sp.at[pl.ds(rd, rpt_d)],
                        denp_r.at[c, pl.ds(rd, rpt_d)])

    return k(src, dst, L, R, F, zacc, zden)


def _att_mat(a, din):
    H, D = a.shape
    m = jnp.zeros((din, 16), F32)
    for h in range(H):
        m = m.at[h * D:(h + 1) * D, h].set(a[h])
    return m


def _ex_mat(dh, H, D):
    m = jnp.zeros((16, dh), F32)
    for h in range(H):
        m = m.at[h, h * D:(h + 1) * D].set(1.0)
    return m


def _pad_rows(x):
    return jnp.pad(x, ((0, NP - N), (0, 0)))


def kernel(edge_index, features, W0, al0, ar0, b0, W1, al1, ar1, b1,
           W2, al2, ar2, b2, resW2):
    ei = edge_index.astype(jnp.int32)
    src, dst = ei[0], ei[1]

    I128 = jnp.eye(128, dtype=F32)
    zn128 = jnp.zeros((N, 128), F32)
    zn16 = jnp.zeros((N, 16), F32)
    zacc8 = jnp.zeros((NP * 8, 16), F32)
    zacc4 = jnp.zeros((NP * 4, 16), F32)
    zden = jnp.zeros((NP, 16), F32)
    z8_128 = jnp.zeros((8, 128), F32)
    ex128 = _ex_mat(128, 8, 16)
    ex64 = _ex_mat(64, 1, 64)
    heads8 = tuple(range(8))
    heads1 = (0, 0, 0, 0)

    _, f0, L0, R0 = _tc_stage(zn128, zn128, zn16, zn16, ex128, features,
                              I128, z8_128, W0,
                              _att_mat(al0, 128), _att_mat(ar0, 128))
    a0, d0 = _sc_pass(src, dst, _pad_rows(L0), _pad_rows(R0),
                      _pad_rows(f0), zacc8, zden, heads8)

    b0t = jnp.tile(b0.reshape(1, 128), (8, 1))
    o0A = a0[0].reshape(NP, 128)[:N]
    o0B = a0[1].reshape(NP, 128)[:N]
    h1, f1, L1, R1 = _tc_stage(o0A, o0B, d0[0, :N, :], d0[1, :N, :],
                               ex128, zn128, I128, b0t, W1,
                               _att_mat(al1, 128), _att_mat(ar1, 128))
    a1, d1 = _sc_pass(src, dst, _pad_rows(L1), _pad_rows(R1),
                      _pad_rows(f1), zacc8, zden, heads8)

    b1t = jnp.tile(b1.reshape(1, 128), (8, 1))
    W2p = jnp.pad(W2, ((0, 0), (0, 24)))
    al2m = jnp.zeros((64, 16), F32).at[0:40, 0].set(al2[0])
    ar2m = jnp.zeros((64, 16), F32).at[0:40, 0].set(ar2[0])
    o1A = a1[0].reshape(NP, 128)[:N]
    o1B = a1[1].reshape(NP, 128)[:N]
    h2, f2, L2, R2 = _tc_stage(o1A, o1B, d1[0, :N, :], d1[1, :N, :],
                               ex128, h1, I128, b1t, W2p, al2m, ar2m)
    a2, d2 = _sc_pass(src, dst, _pad_rows(L2), _pad_rows(R2),
                      _pad_rows(f2), zacc4, zden, heads1)

    resW2p = jnp.pad(resW2, ((0, 0), (0, 24)))
    b2t = jnp.tile(jnp.pad(b2, (0, 24)).reshape(1, 64), (8, 1))
    o2A = a2[0].reshape(NP, 64)[:N]
    o2B = a2[1].reshape(NP, 64)[:N]
    (h3,) = _tc_stage(o2A, o2B, d2[0, :N, :], d2[1, :N, :],
                      ex64, h2, resW2p, b2t)
    return h3[:, :40]


# speedup vs baseline: 37.3094x; 1.0326x over previous
import functools

import jax
import jax.numpy as jnp
from jax import lax
from jax.experimental import pallas as pl
from jax.experimental.pallas import tpu as pltpu
from jax.experimental.pallas import tpu_sc as plsc

N = 10000
E = 320000
NP = 10240
NC = 2
NS = 16
NW = NC * NS
EPW = E // NW
CH = 48
NCHUNK = -(-EPW // CH)
EPWP = NCHUNK * CH
F32 = jnp.float32


def _tc_stage(outA, outB, denA, denB, ex, resin, wres, bias8, W=None,
              aal=None, aar=None):
    with_proj = W is not None
    dh = outA.shape[1]
    dr = resin.shape[1]
    B = 400
    grid = (N // B,)

    def body(*refs):
        if with_proj:
            (oA, oB, dA, dB, exr, rin, wr, br, Wr, alr, arr,
             h_o, f_o, l_o, r_o) = refs
        else:
            oA, oB, dA, dB, exr, rin, wr, br, h_o = refs
        den = dA[...] + dB[...]
        rec = 1.0 / jnp.maximum(den, 1e-9)
        recx = jnp.dot(rec, exr[...], preferred_element_type=F32)
        h = (oA[...] + oB[...]) * recx
        h = h + jnp.dot(rin[...], wr[...], preferred_element_type=F32)
        h = h + br[0:1, :]
        h_o[...] = h
        if with_proj:
            f = jnp.dot(h, Wr[...], preferred_element_type=F32)
            f_o[...] = f
            l_o[...] = jnp.dot(f, alr[...], preferred_element_type=F32)
            r_o[...] = jnp.dot(f, arr[...], preferred_element_type=F32)

    node = lambda d: pl.BlockSpec((B, d), lambda i: (i, 0))
    full = lambda a: pl.BlockSpec(a.shape, lambda i: (0, 0))
    in_specs = [node(dh), node(dh), node(16), node(16), full(ex),
                node(dr), full(wres), full(bias8)]
    args = [outA, outB, denA, denB, ex, resin, wres, bias8]
    out_shapes = [jax.ShapeDtypeStruct((N, dh), F32)]
    out_specs = [node(dh)]
    if with_proj:
        dn = W.shape[1]
        in_specs += [full(W), full(aal), full(aar)]
        args += [W, aal, aar]
        out_shapes += [jax.ShapeDtypeStruct((N, dn), F32),
                       jax.ShapeDtypeStruct((N, 16), F32),
                       jax.ShapeDtypeStruct((N, 16), F32)]
        out_specs += [node(dn), node(16), node(16)]
    return pl.pallas_call(
        body, grid=grid, in_specs=in_specs, out_specs=out_specs,
        out_shape=out_shapes)(*args)


def _sc_pass(esd, L, R, F, zacc, zden, headmap):
    DW = F.shape[1]
    npiece = DW // 16
    lpe = npiece
    tr = CH * npiece
    blen = {8: 128, 4: 80}[npiece]
    nbatch = tr // blen
    vpb = blen // 16
    epv = 16 // npiece
    rpt_a = NP * npiece // NS
    rpt_d = NP // NS
    mesh = plsc.VectorSubcoreMesh(core_axis_name="c", subcore_axis_name="s")

    @functools.partial(
        pl.kernel,
        out_type=[jax.ShapeDtypeStruct((NC, NP * npiece, 16), F32),
                  jax.ShapeDtypeStruct((NC, NP, 16), F32)],
        mesh=mesh,
        compiler_params=pltpu.CompilerParams(use_tc_tiling_on_sc=False),
        scratch_types=[
            pltpu.VMEM_SHARED((NP * npiece, 16), F32),
            pltpu.VMEM_SHARED((NP, 16), F32),
            [pltpu.VMEM((2 * CH,), jnp.int32) for _ in range(2)],
            [pltpu.VMEM((CH, 16), F32) for _ in range(2)],
            [pltpu.VMEM((CH, 16), F32) for _ in range(2)],
            [pltpu.VMEM((CH, DW), F32) for _ in range(2)],
            [pltpu.VMEM((CH, 16), F32) for _ in range(2)],
            [pltpu.VMEM((tr, 16), F32) for _ in range(2)],
            [pltpu.VMEM((tr,), jnp.int32) for _ in range(2)],
            [pltpu.VMEM((CH,), jnp.int32) for _ in range(2)],
            [pltpu.SemaphoreType.DMA for _ in range(2)],
            [pltpu.SemaphoreType.DMA for _ in range(2)],
        ],
    )
    def k(esd_r, L_r, R_r, F_r, zacc_r, zden_r, accp_r, denp_r,
          acc_sp, den_sp, sds, lrows, rrows, frows, wrows, fss,
          idxbs, dios, semg, semsc):
        c = lax.axis_index("c")
        s = lax.axis_index("s")
        g = s * NC + c
        ra = s * rpt_a
        rd = s * rpt_d
        pltpu.sync_copy(zacc_r.at[pl.ds(ra, rpt_a)],
                        acc_sp.at[pl.ds(ra, rpt_a)])
        pltpu.sync_copy(zden_r.at[pl.ds(rd, rpt_d)],
                        den_sp.at[pl.ds(rd, rpt_d)])
        plsc.subcore_barrier()

        nchunk = NCHUNK
        base_cid = g * nchunk
        io = lax.iota(jnp.int32, 16)
        jadd = io & (npiece - 1)

        def load_idx(i, b):
            off = (base_cid + i) * (2 * CH)
            pltpu.sync_copy(esd_r.at[pl.ds(off, 2 * CH)], sds[b])

        def g_copies(b, make):
            si = sds[b].at[pl.ds(0, CH)]
            di = sds[b].at[pl.ds(CH, CH)]
            return [make(L_r.at[si], lrows[b], semg[b]),
                    make(R_r.at[di], rrows[b], semg[b]),
                    make(F_r.at[si], frows[b], semg[b])]

        def sc_copies(b, make):
            return [make(wrows[b], den_sp.at[dios[b]], semsc[b]),
                    make(fss[b], acc_sp.at[idxbs[b]], semsc[b])]

        def start_g(b):
            g_copies(b, lambda s_, d_, m_: pltpu.async_copy(s_, d_, m_))

        def wait_g(b):
            for cp in g_copies(b, pltpu.make_async_copy):
                cp.wait()

        def issue_sc(b):
            sc_copies(b, lambda s_, d_, m_:
                      pltpu.async_copy(s_, d_, m_, add=True))

        def drain_sc(b):
            for cp in sc_copies(b, pltpu.make_async_copy):
                cp.wait()

        def process(b):
            lrow, rrow, frow = lrows[b], rrows[b], frows[b]
            wrow, fs, idxb = wrows[b], fss[b], idxbs[b]
            drain_sc(b)
            dvs = [sds[b][pl.ds(CH + 16 * q, 16)] for q in range(CH // 16)]
            for q in range(CH // 16):
                dios[b][pl.ds(16 * q, 16)] = dvs[q]
            for v in range(tr // 16):
                e0 = v * epv
                dv = dvs[e0 // 16]
                lane0 = e0 % 16
                d = dv[lane0]
                for t in range(1, epv):
                    d = jnp.where(io < t * lpe, d, dv[lane0 + t])
                vec = d * npiece + jadd
                idxb[pl.ds(16 * v, 16)] = vec

            @pl.loop(0, CH, unroll=4)
            def _edge(e):
                x = lrow[e, :] + rrow[e, :]
                x = jnp.maximum(x, x * 0.2)
                w = jnp.exp(x)
                wrow[e, :] = w
                for j, hj in enumerate(headmap):
                    fs[e * npiece + j, :] = frow[e, pl.ds(16 * j, 16)] * w[hj]

            issue_sc(b)

        for b in range(2):
            pltpu.sync_copy(zacc_r.at[pl.ds(0, tr)], fss[b])
            pltpu.sync_copy(zden_r.at[pl.ds(0, CH)], wrows[b])
            pltpu.sync_copy(esd_r.at[pl.ds(0, tr)], idxbs[b])
            pltpu.sync_copy(esd_r.at[pl.ds(0, CH)], dios[b])
            issue_sc(b)

        load_idx(0, 0)
        start_g(0)

        @pl.loop(0, nchunk - 1, step=2)
        def _pair(i):
            load_idx(i + 1, 1)
            start_g(1)
            wait_g(0)
            process(0)
            load_idx(i + 2, 0)
            start_g(0)
            wait_g(1)
            process(1)

        wait_g(0)
        process(0)
        drain_sc(0)
        drain_sc(1)

        plsc.subcore_barrier()
        pltpu.sync_copy(acc_sp.at[pl.ds(ra, rpt_a)],
                        accp_r.at[c, pl.ds(ra, rpt_a)])
        pltpu.sync_copy(den_sp.at[pl.ds(rd, rpt_d)],
                        denp_r.at[c, pl.ds(rd, rpt_d)])

    return k(esd, L, R, F, zacc, zden)


def _att_mat(a, din):
    H, D = a.shape
    m = jnp.zeros((din, 16), F32)
    for h in range(H):
        m = m.at[h * D:(h + 1) * D, h].set(a[h])
    return m


def _ex_mat(dh, H, D):
    m = jnp.zeros((16, dh), F32)
    for h in range(H):
        m = m.at[h, h * D:(h + 1) * D].set(1.0)
    return m


def _pad_rows(x):
    return jnp.pad(x, ((0, NP - N), (0, 0)))


def kernel(edge_index, features, W0, al0, ar0, b0, W1, al1, ar1, b1,
           W2, al2, ar2, b2, resW2):
    ei = edge_index.astype(jnp.int32)
    src, dst = ei[0], ei[1]
    pad = jnp.full((NW, EPWP - EPW), NP - 1, jnp.int32)
    srcp = jnp.concatenate([src.reshape(NW, EPW), pad], axis=1)
    dstp = jnp.concatenate([dst.reshape(NW, EPW), pad], axis=1)
    esd = jnp.stack([srcp.reshape(NW, NCHUNK, CH),
                     dstp.reshape(NW, NCHUNK, CH)], axis=2).reshape(-1)

    I128 = jnp.eye(128, dtype=F32)
    zn128 = jnp.zeros((N, 128), F32)
    zn16 = jnp.zeros((N, 16), F32)
    zacc8 = jnp.zeros((NP * 8, 16), F32)
    zacc4 = jnp.zeros((NP * 4, 16), F32)
    zden = jnp.zeros((NP, 16), F32)
    z8_128 = jnp.zeros((8, 128), F32)
    ex128 = _ex_mat(128, 8, 16)
    ex64 = _ex_mat(64, 1, 64)
    heads8 = tuple(range(8))
    heads1 = (0, 0, 0, 0)

    _, f0, L0, R0 = _tc_stage(zn128, zn128, zn16, zn16, ex128, features,
                              I128, z8_128, W0,
                              _att_mat(al0, 128), _att_mat(ar0, 128))
    a0, d0 = _sc_pass(esd, _pad_rows(L0), _pad_rows(R0),
                      _pad_rows(f0), zacc8, zden, heads8)

    b0t = jnp.tile(b0.reshape(1, 128), (8, 1))
    o0A = a0[0].reshape(NP, 128)[:N]
    o0B = a0[1].reshape(NP, 128)[:N]
    h1, f1, L1, R1 = _tc_stage(o0A, o0B, d0[0, :N, :], d0[1, :N, :],
                               ex128, zn128, I128, b0t, W1,
                               _att_mat(al1, 128), _att_mat(ar1, 128))
    a1, d1 = _sc_pass(esd, _pad_rows(L1), _pad_rows(R1),
                      _pad_rows(f1), zacc8, zden, heads8)

    b1t = jnp.tile(b1.reshape(1, 128), (8, 1))
    W2p = jnp.pad(W2, ((0, 0), (0, 24)))
    al2m = jnp.zeros((64, 16), F32).at[0:40, 0].set(al2[0])
    ar2m = jnp.zeros((64, 16), F32).at[0:40, 0].set(ar2[0])
    o1A = a1[0].reshape(NP, 128)[:N]
    o1B = a1[1].reshape(NP, 128)[:N]
    h2, f2, L2, R2 = _tc_stage(o1A, o1B, d1[0, :N, :], d1[1, :N, :],
                               ex128, h1, I128, b1t, W2p, al2m, ar2m)
    a2, d2 = _sc_pass(esd, _pad_rows(L2), _pad_rows(R2),
                      _pad_rows(f2), zacc4, zden, heads1)

    resW2p = jnp.pad(resW2, ((0, 0), (0, 24)))
    b2t = jnp.tile(jnp.pad(b2, (0, 24)).reshape(1, 64), (8, 1))
    o2A = a2[0].reshape(NP, 64)[:N]
    o2B = a2[1].reshape(NP, 64)[:N]
    (h3,) = _tc_stage(o2A, o2B, d2[0, :N, :], d2[1, :N, :],
                      ex64, h2, resW2p, b2t)
    return h3[:, :40]


# speedup vs baseline: 37.3215x; 1.0003x over previous
import functools

import jax
import jax.numpy as jnp
from jax import lax
from jax.experimental import pallas as pl
from jax.experimental.pallas import tpu as pltpu
from jax.experimental.pallas import tpu_sc as plsc

N = 10000
E = 320000
NP = 10240
NC = 2
NS = 16
NW = NC * NS
EPW = E // NW
CH = 48
NCHUNK = -(-EPW // CH)
EPWP = NCHUNK * CH
F32 = jnp.float32


def _tc_stage(outA, outB, denA, denB, ex, resin, wres, bias8, W=None,
              aal=None, aar=None):
    with_proj = W is not None
    dh = outA.shape[1]
    dr = resin.shape[1]
    B = 400
    grid = (N // B,)

    def body(*refs):
        if with_proj:
            (oA, oB, dA, dB, exr, rin, wr, br, Wr, alr, arr,
             h_o, f_o, l_o, r_o) = refs
        else:
            oA, oB, dA, dB, exr, rin, wr, br, h_o = refs
        den = dA[...] + dB[...]
        rec = 1.0 / jnp.maximum(den, 1e-9)
        recx = jnp.dot(rec, exr[...], preferred_element_type=F32)
        h = (oA[...] + oB[...]) * recx
        h = h + jnp.dot(rin[...], wr[...], preferred_element_type=F32)
        h = h + br[0:1, :]
        h_o[...] = h
        if with_proj:
            f = jnp.dot(h, Wr[...], preferred_element_type=F32)
            f_o[...] = f
            l_o[...] = jnp.dot(f, alr[...], preferred_element_type=F32)
            r_o[...] = jnp.dot(f, arr[...], preferred_element_type=F32)

    node = lambda d: pl.BlockSpec((B, d), lambda i: (i, 0))
    full = lambda a: pl.BlockSpec(a.shape, lambda i: (0, 0))
    in_specs = [node(dh), node(dh), node(16), node(16), full(ex),
                node(dr), full(wres), full(bias8)]
    args = [outA, outB, denA, denB, ex, resin, wres, bias8]
    out_shapes = [jax.ShapeDtypeStruct((N, dh), F32)]
    out_specs = [node(dh)]
    if with_proj:
        dn = W.shape[1]
        in_specs += [full(W), full(aal), full(aar)]
        args += [W, aal, aar]
        out_shapes += [jax.ShapeDtypeStruct((N, dn), F32),
                       jax.ShapeDtypeStruct((N, 16), F32),
                       jax.ShapeDtypeStruct((N, 16), F32)]
        out_specs += [node(dn), node(16), node(16)]
    return pl.pallas_call(
        body, grid=grid, in_specs=in_specs, out_specs=out_specs,
        out_shape=out_shapes)(*args)


def _sc_pass(esd, L, R, F, zacc, zden, headmap):
    DW = F.shape[1]
    npiece = DW // 16
    lpe = npiece
    tr = CH * npiece
    blen = {8: 128, 4: 80}[npiece]
    nbatch = tr // blen
    vpb = blen // 16
    epv = 16 // npiece
    rpt_a = NP * npiece // NS
    rpt_d = NP // NS
    mesh = plsc.VectorSubcoreMesh(core_axis_name="c", subcore_axis_name="s")

    @functools.partial(
        pl.kernel,
        out_type=[jax.ShapeDtypeStruct((NC, NP * npiece, 16), F32),
                  jax.ShapeDtypeStruct((NC, NP, 16), F32)],
        mesh=mesh,
        compiler_params=pltpu.CompilerParams(use_tc_tiling_on_sc=False),
        scratch_types=[
            pltpu.VMEM_SHARED((NP * npiece, 16), F32),
            pltpu.VMEM_SHARED((NP, 16), F32),
            [pltpu.VMEM((2 * CH,), jnp.int32) for _ in range(2)],
            [pltpu.VMEM((CH, 16), F32) for _ in range(2)],
            [pltpu.VMEM((CH, 16), F32) for _ in range(2)],
            [pltpu.VMEM((CH, DW), F32) for _ in range(2)],
            [pltpu.VMEM((CH, 16), F32) for _ in range(2)],
            [pltpu.VMEM((tr, 16), F32) for _ in range(2)],
            [pltpu.VMEM((tr,), jnp.int32) for _ in range(2)],
            [pltpu.VMEM((CH,), jnp.int32) for _ in range(2)],
            [pltpu.SemaphoreType.DMA for _ in range(2)],
            [pltpu.SemaphoreType.DMA for _ in range(2)],
        ],
    )
    def k(esd_r, L_r, R_r, F_r, zacc_r, zden_r, accp_r, denp_r,
          acc_sp, den_sp, sds, lrows, rrows, frows, wrows, fss,
          idxbs, dios, semg, semsc):
        c = lax.axis_index("c")
        s = lax.axis_index("s")
        g = s * NC + c
        ra = s * rpt_a
        rd = s * rpt_d
        pltpu.sync_copy(zacc_r.at[pl.ds(ra, rpt_a)],
                        acc_sp.at[pl.ds(ra, rpt_a)])
        pltpu.sync_copy(zden_r.at[pl.ds(rd, rpt_d)],
                        den_sp.at[pl.ds(rd, rpt_d)])
        plsc.subcore_barrier()

        nchunk = NCHUNK
        base_cid = g * nchunk
        io = lax.iota(jnp.int32, 16)
        jadd = io & (npiece - 1)

        def load_idx(i, b):
            off = (base_cid + i) * (2 * CH)
            pltpu.sync_copy(esd_r.at[pl.ds(off, 2 * CH)], sds[b])

        def g_copies(b, make):
            si = sds[b].at[pl.ds(0, CH)]
            di = sds[b].at[pl.ds(CH, CH)]
            return [make(L_r.at[si], lrows[b], semg[b]),
                    make(R_r.at[di], rrows[b], semg[b]),
                    make(F_r.at[si], frows[b], semg[b])]

        def sc_copies(b, make):
            return [make(wrows[b], den_sp.at[dios[b]], semsc[b]),
                    make(fss[b], acc_sp.at[idxbs[b]], semsc[b])]

        def start_g(b):
            g_copies(b, lambda s_, d_, m_: pltpu.async_copy(s_, d_, m_))

        def wait_g(b):
            for cp in g_copies(b, pltpu.make_async_copy):
                cp.wait()

        def issue_sc(b):
            sc_copies(b, lambda s_, d_, m_:
                      pltpu.async_copy(s_, d_, m_, add=True))

        def drain_sc(b):
            for cp in sc_copies(b, pltpu.make_async_copy):
                cp.wait()

        def process(b):
            lrow, rrow, frow = lrows[b], rrows[b], frows[b]
            wrow, fs, idxb = wrows[b], fss[b], idxbs[b]
            drain_sc(b)
            dvs = [sds[b][pl.ds(CH + 16 * q, 16)] for q in range(CH // 16)]
            for q in range(CH // 16):
                dios[b][pl.ds(16 * q, 16)] = dvs[q]
            for v in range(tr // 16):
                e0 = v * epv
                dv = dvs[e0 // 16]
                lane0 = e0 % 16
                d = dv[lane0]
                for t in range(1, epv):
                    d = jnp.where(io < t * lpe, d, dv[lane0 + t])
                vec = d * npiece + jadd
                idxb[pl.ds(16 * v, 16)] = vec

            @pl.loop(0, CH, unroll=8)
            def _edge(e):
                x = lrow[e, :] + rrow[e, :]
                x = jnp.maximum(x, x * 0.2)
                w = jnp.exp(x)
                wrow[e, :] = w
                for j, hj in enumerate(headmap):
                    fs[e * npiece + j, :] = frow[e, pl.ds(16 * j, 16)] * w[hj]

            issue_sc(b)

        for b in range(2):
            pltpu.sync_copy(zacc_r.at[pl.ds(0, tr)], fss[b])
            pltpu.sync_copy(zden_r.at[pl.ds(0, CH)], wrows[b])
            pltpu.sync_copy(esd_r.at[pl.ds(0, tr)], idxbs[b])
            pltpu.sync_copy(esd_r.at[pl.ds(0, CH)], dios[b])
            issue_sc(b)

        load_idx(0, 0)
        start_g(0)

        @pl.loop(0, nchunk - 1, step=2)
        def _pair(i):
            load_idx(i + 1, 1)
            start_g(1)
            wait_g(0)
            process(0)
            load_idx(i + 2, 0)
            start_g(0)
            wait_g(1)
            process(1)

        wait_g(0)
        process(0)
        drain_sc(0)
        drain_sc(1)

        plsc.subcore_barrier()
        pltpu.sync_copy(acc_sp.at[pl.ds(ra, rpt_a)],
                        accp_r.at[c, pl.ds(ra, rpt_a)])
        pltpu.sync_copy(den_sp.at[pl.ds(rd, rpt_d)],
                        denp_r.at[c, pl.ds(rd, rpt_d)])

    return k(esd, L, R, F, zacc, zden)


def _att_mat(a, din):
    H, D = a.shape
    m = jnp.zeros((din, 16), F32)
    for h in range(H):
        m = m.at[h * D:(h + 1) * D, h].set(a[h])
    return m


def _ex_mat(dh, H, D):
    m = jnp.zeros((16, dh), F32)
    for h in range(H):
        m = m.at[h, h * D:(h + 1) * D].set(1.0)
    return m


def _pad_rows(x):
    return jnp.pad(x, ((0, NP - N), (0, 0)))


def kernel(edge_index, features, W0, al0, ar0, b0, W1, al1, ar1, b1,
           W2, al2, ar2, b2, resW2):
    ei = edge_index.astype(jnp.int32)
    src, dst = ei[0], ei[1]
    pad = jnp.full((NW, EPWP - EPW), NP - 1, jnp.int32)
    srcp = jnp.concatenate([src.reshape(NW, EPW), pad], axis=1)
    dstp = jnp.concatenate([dst.reshape(NW, EPW), pad], axis=1)
    esd = jnp.stack([srcp.reshape(NW, NCHUNK, CH),
                     dstp.reshape(NW, NCHUNK, CH)], axis=2).reshape(-1)

    I128 = jnp.eye(128, dtype=F32)
    zn128 = jnp.zeros((N, 128), F32)
    zn16 = jnp.zeros((N, 16), F32)
    zacc8 = jnp.zeros((NP * 8, 16), F32)
    zacc4 = jnp.zeros((NP * 4, 16), F32)
    zden = jnp.zeros((NP, 16), F32)
    z8_128 = jnp.zeros((8, 128), F32)
    ex128 = _ex_mat(128, 8, 16)
    ex64 = _ex_mat(64, 1, 64)
    heads8 = tuple(range(8))
    heads1 = (0, 0, 0, 0)

    _, f0, L0, R0 = _tc_stage(zn128, zn128, zn16, zn16, ex128, features,
                              I128, z8_128, W0,
                              _att_mat(al0, 128), _att_mat(ar0, 128))
    a0, d0 = _sc_pass(esd, _pad_rows(L0), _pad_rows(R0),
                      _pad_rows(f0), zacc8, zden, heads8)

    b0t = jnp.tile(b0.reshape(1, 128), (8, 1))
    o0A = a0[0].reshape(NP, 128)[:N]
    o0B = a0[1].reshape(NP, 128)[:N]
    h1, f1, L1, R1 = _tc_stage(o0A, o0B, d0[0, :N, :], d0[1, :N, :],
                               ex128, zn128, I128, b0t, W1,
                               _att_mat(al1, 128), _att_mat(ar1, 128))
    a1, d1 = _sc_pass(esd, _pad_rows(L1), _pad_rows(R1),
                      _pad_rows(f1), zacc8, zden, heads8)

    b1t = jnp.tile(b1.reshape(1, 128), (8, 1))
    W2p = jnp.pad(W2, ((0, 0), (0, 24)))
    al2m = jnp.zeros((64, 16), F32).at[0:40, 0].set(al2[0])
    ar2m = jnp.zeros((64, 16), F32).at[0:40, 0].set(ar2[0])
    o1A = a1[0].reshape(NP, 128)[:N]
    o1B = a1[1].reshape(NP, 128)[:N]
    h2, f2, L2, R2 = _tc_stage(o1A, o1B, d1[0, :N, :], d1[1, :N, :],
                               ex128, h1, I128, b1t, W2p, al2m, ar2m)
    a2, d2 = _sc_pass(esd, _pad_rows(L2), _pad_rows(R2),
                      _pad_rows(f2), zacc4, zden, heads1)

    resW2p = jnp.pad(resW2, ((0, 0), (0, 24)))
    b2t = jnp.tile(jnp.pad(b2, (0, 24)).reshape(1, 64), (8, 1))
    o2A = a2[0].reshape(NP, 64)[:N]
    o2B = a2[1].reshape(NP, 64)[:N]
    (h3,) = _tc_stage(o2A, o2B, d2[0, :N, :], d2[1, :N, :],
                      ex64, h2, resW2p, b2t)
    return h3[:, :40]
